# Initial kernel scaffold; baseline (speedup 1.0000x reference)
import jax
import jax.numpy as jnp
from jax.experimental import pallas as pl


def kernel(token_ids, W):
    raise NotImplementedError("write your pallas kernel here")


# speedup vs baseline: 16.6448x; 16.6448x over previous
import functools

import jax
import jax.numpy as jnp
from jax import lax
from jax.experimental import pallas as pl
from jax.experimental.pallas import tpu as pltpu
from jax.experimental.pallas import tpu_sc as plsc

_NW = 32
_L = 16


@functools.partial(jax.jit, static_argnames=("n", "chunk"))
def _sc_gather(flat_ids, table8, n, chunk):
    per_w = n // _NW
    steps = per_w // chunk
    groups = chunk // _L

    mesh = plsc.VectorSubcoreMesh(core_axis_name="c", subcore_axis_name="s")

    @functools.partial(
        pl.kernel,
        out_type=jax.ShapeDtypeStruct((n, 2), jnp.float32),
        mesh=mesh,
        scratch_types=[
            pltpu.VMEM((chunk,), jnp.int32),
            pltpu.VMEM((chunk,), jnp.int32),
            pltpu.VMEM((chunk, 8), jnp.float32),
            pltpu.VMEM((chunk, 2), jnp.float32),
            pltpu.SemaphoreType.DMA,
        ],
        compiler_params=pltpu.CompilerParams(
            use_tc_tiling_on_sc=False, needs_layout_passes=False
        ),
    )
    def body(ids_hbm, tab_hbm, out_hbm, idx_v, line_v, rows_v, comp_v, sem):
        wid = lax.axis_index("s") * 2 + lax.axis_index("c")
        base = wid * per_w
        iota = lax.iota(jnp.int32, _L)
        zeros = jnp.zeros((_L,), jnp.int32)
        ones = zeros + 1

        def step(s, carry):
            off = base + s * chunk
            pltpu.sync_copy(ids_hbm.at[pl.ds(off, chunk)], idx_v)

            def lines(g, carry2):
                v = idx_v[pl.ds(g * _L, _L)]
                line_v[pl.ds(g * _L, _L)] = lax.shift_right_logical(v, 2)
                return carry2

            lax.fori_loop(0, groups, lines, 0)

            pltpu.async_copy(tab_hbm.at[line_v], rows_v, sem).wait()

            def extract(g, carry2):
                r16 = iota + g * _L
                v = idx_v[pl.ds(g * _L, _L)]
                col = lax.shift_left(v & 3, 1)
                x = plsc.load_gather(rows_v, [r16, col])
                y = plsc.load_gather(rows_v, [r16, col + 1])
                plsc.store_scatter(comp_v, [r16, zeros], x)
                plsc.store_scatter(comp_v, [r16, ones], y)
                return carry2

            lax.fori_loop(0, groups, extract, 0)

            pltpu.sync_copy(comp_v, out_hbm.at[pl.ds(off, chunk)])
            return carry

        lax.fori_loop(0, steps, step, 0)

    return body(flat_ids, table8)


def kernel(token_ids, W):
    b, h = token_ids.shape
    n = b * h
    flat = token_ids.reshape(n).astype(jnp.int32)
    table8 = W.reshape(W.shape[0] // 4, 8)
    out = _sc_gather(flat, table8, n, 2048)
    return out.reshape(b, h, 2)
